# baseline (speedup 1.0000x reference)
import functools
import math

import jax
import jax.numpy as jnp
from jax import lax
from jax.experimental import pallas as pl
from jax.experimental.pallas import tpu as pltpu
from jax.experimental.pallas import tpu_sc as plsc

D_MODEL = 64
SCALE = math.sqrt(D_MODEL)
LANES = 16

_info = plsc.get_sparse_core_info()
NUM_CORES = _info.num_cores
NUM_SUBCORES = _info.num_subcores
NUM_WORKERS = NUM_CORES * NUM_SUBCORES

RBLK = 128
SUBL = 8
NSLOT = 4
INFLIGHT = 2
UNROLL = 8


def _make_lookup(n_rows, row_len, d_model):
    assert n_rows == NUM_WORKERS * RBLK
    n_chunks = row_len
    n_jt = d_model // SUBL
    assert n_chunks % UNROLL == 0 and n_chunks // UNROLL >= 2
    n_ct = n_chunks // UNROLL
    assert d_model % LANES == 0

    mesh = plsc.VectorSubcoreMesh(core_axis_name="c", subcore_axis_name="s")

    @functools.partial(
        pl.kernel,
        mesh=mesh,
        compiler_params=pltpu.CompilerParams(use_tc_tiling_on_sc=False,
                                             needs_layout_passes=False),
        out_type=jax.ShapeDtypeStruct(
            (n_chunks, n_jt, NUM_WORKERS, SUBL, RBLK), jnp.float32),
        scratch_types=[
            pltpu.VMEM((n_chunks, RBLK), jnp.int32),
            pltpu.VMEM((NSLOT, RBLK, d_model), jnp.float32),
            pltpu.VMEM((NSLOT, d_model, RBLK), jnp.float32),
            pltpu.SemaphoreType.DMA((NSLOT,)),
            pltpu.SemaphoreType.DMA((NSLOT,)),
        ],
    )
    def lookup(x_hbm, table_hbm, out_hbm, idx_v, rows_v, stage_v, gsem, ssem):
        wid = lax.axis_index("s") * NUM_CORES + lax.axis_index("c")

        pltpu.sync_copy(x_hbm.at[:, pl.ds(wid * RBLK, RBLK)], idx_v)

        def gather_start(g, s):
            pltpu.async_copy(table_hbm.at[idx_v.at[g]], rows_v.at[s],
                             gsem.at[s])

        def gather_wait(g, s):
            pltpu.make_async_copy(table_hbm.at[idx_v.at[g]], rows_v.at[s],
                                  gsem.at[s]).wait()

        def scatter_start(g, s):
            for jt in range(n_jt):
                pltpu.async_copy(stage_v.at[s, pl.ds(jt * SUBL, SUBL)],
                                 out_hbm.at[g, jt, wid], ssem.at[s])

        def scatter_wait(g, s):
            for jt in range(n_jt):
                pltpu.make_async_copy(stage_v.at[s, pl.ds(jt * SUBL, SUBL)],
                                      out_hbm.at[g, jt, wid],
                                      ssem.at[s]).wait()

        iota = lax.iota(jnp.int32, LANES)
        rowidx = [iota + lg * LANES for lg in range(RBLK // LANES)]

        def scale(s):
            def diag_body(t, c):
                db = t // LANES
                k = t % LANES
                dv = ((iota + k) & (LANES - 1)) + db * LANES
                for lg in range(RBLK // LANES):
                    v = plsc.load_gather(rows_v.at[s], [rowidx[lg], dv])
                    plsc.store_scatter(stage_v.at[s], [dv, rowidx[lg]],
                                       v * SCALE)
                return c
            lax.fori_loop(0, d_model, diag_body, 0)

        def step(g, j, first_ct, last_ct):
            s = j % NSLOT
            gather_wait(g, s)
            scale(s)
            scatter_start(g, s)
            pre = g + INFLIGHT
            sp = (j + INFLIGHT) % NSLOT
            if last_ct and j >= UNROLL - INFLIGHT:
                return
            if not (first_ct and j < NSLOT - INFLIGHT):
                scatter_wait(pre - NSLOT, sp)
            gather_start(pre, sp)

        for b in range(INFLIGHT):
            gather_start(b, b)

        for j in range(UNROLL):
            step(j, j, True, False)

        def ct_body(o, carry):
            for j in range(UNROLL):
                step(o * UNROLL + j, j, False, False)
            return carry

        lax.fori_loop(1, n_ct - 1, ct_body, 0)

        g0 = (n_ct - 1) * UNROLL
        for j in range(UNROLL):
            step(g0 + j, j, False, True)
        for j in range(UNROLL - NSLOT, UNROLL):
            scatter_wait(g0 + j, j % NSLOT)

    return lookup


def kernel(x, table):
    n_rows, row_len = x.shape
    d_model = table.shape[1]
    x_t = jnp.swapaxes(x, 0, 1).astype(jnp.int32)
    out5 = _make_lookup(n_rows, row_len, d_model)(x_t, table)
    return out5.transpose(2, 4, 0, 1, 3).reshape(n_rows, row_len, d_model)
